# Initial kernel scaffold; baseline (speedup 1.0000x reference)
import jax
import jax.numpy as jnp
from jax.experimental import pallas as pl


def kernel(x, edge_index, edge_attr, W1n, b1n, W2n, b2n, W1e, b1e, W2e, b2e, gcn_W, gcn_b, Wp1, bp1, Wp2, bp2, Wr, br):
    raise NotImplementedError("write your pallas kernel here")


# speedup vs baseline: 1.8894x; 1.8894x over previous
import functools
import jax
import jax.numpy as jnp
from jax.experimental import pallas as pl
from jax.experimental.pallas import tpu as pltpu

N = 50000
NB = 512
NPAD = 50176


def _leaky(v, s):
    return jnp.where(v >= 0, v, s * v)


def _pre_kernel(x_ref, W1n_ref, b1n_ref, W2n_ref, b2n_ref, o_ref):
    x = jnp.nan_to_num(x_ref[...], nan=0.0)
    h = _leaky(jnp.dot(x, W1n_ref[...], preferred_element_type=jnp.float32)
               + b1n_ref[...][None, :], 0.2)
    h = _leaky(jnp.dot(h, W2n_ref[...], preferred_element_type=jnp.float32)
               + b2n_ref[...][None, :], 0.2)
    o_ref[...] = h


def _post_kernel(h_ref, Wp1_ref, bp1_ref, Wp2_ref, bp2_ref, Wr_ref, br_ref, o_ref):
    h = _leaky(jnp.dot(h_ref[...], Wp1_ref[...], preferred_element_type=jnp.float32)
               + bp1_ref[...][None, :], 0.2)
    h = _leaky(jnp.dot(h, Wp2_ref[...], preferred_element_type=jnp.float32)
               + bp2_ref[...][None, :], 0.2)
    o_ref[...] = jnp.dot(h, Wr_ref[...], preferred_element_type=jnp.float32) \
        + br_ref[...][None, :]


def _edge_kernel(ea_ref, W1e_ref, b1e_ref, W2e_ref, b2e_ref, o_ref):
    ea = jnp.nan_to_num(ea_ref[...], nan=0.0)
    h = _leaky(jnp.dot(ea, W1e_ref[...], preferred_element_type=jnp.float32)
               + b1e_ref[...][None, :], 0.2)
    w = _leaky(jnp.dot(h, W2e_ref[...], preferred_element_type=jnp.float32)
               + b2e_ref[...][None, :], 0.005)
    o_ref[...] = w


def _row_block(i):
    return (i, 0)


def kernel(x, edge_index, edge_attr, W1n, b1n, W2n, b2n, W1e, b1e, W2e, b2e,
           gcn_W, gcn_b, Wp1, bp1, Wp2, bp2, Wr, br):
    E = edge_index.shape[1]
    xpad = jnp.zeros((NPAD, 7), jnp.float32).at[:N].set(x)

    node = pl.pallas_call(
        _pre_kernel,
        grid=(NPAD // NB,),
        in_specs=[
            pl.BlockSpec((NB, 7), _row_block),
            pl.BlockSpec((7, 64), lambda i: (0, 0)),
            pl.BlockSpec((64,), lambda i: (0,)),
            pl.BlockSpec((64, 64), lambda i: (0, 0)),
            pl.BlockSpec((64,), lambda i: (0,)),
        ],
        out_specs=pl.BlockSpec((NB, 64), _row_block),
        out_shape=jax.ShapeDtypeStruct((NPAD, 64), jnp.float32),
    )(xpad, W1n, b1n, W2n, b2n)[:N]

    EB = 2048
    EPAD = ((E + EB - 1) // EB) * EB
    eapad = jnp.zeros((EPAD, 4), jnp.float32).at[:E].set(edge_attr)
    ew = pl.pallas_call(
        _edge_kernel,
        grid=(EPAD // EB,),
        in_specs=[
            pl.BlockSpec((EB, 4), _row_block),
            pl.BlockSpec((4, 16), lambda i: (0, 0)),
            pl.BlockSpec((16,), lambda i: (0,)),
            pl.BlockSpec((16, 1), lambda i: (0, 0)),
            pl.BlockSpec((1,), lambda i: (0,)),
        ],
        out_specs=pl.BlockSpec((EB, 1), _row_block),
        out_shape=jax.ShapeDtypeStruct((EPAD, 1), jnp.float32),
    )(eapad, W1e, b1e, W2e, b2e)[:E, 0]

    row, col = edge_index[0], edge_index[1]
    deg = jnp.zeros((N,), jnp.float32).at[col].add(ew) + 1.0
    dinv = deg ** -0.5
    dinv = jnp.where(jnp.isinf(dinv), 0.0, dinv)

    for i in range(8):
        xw = node @ gcn_W[i]
        y = dinv[:, None] * xw
        acc = jnp.zeros_like(y).at[col].add(y[row] * ew[:, None])
        node = _leaky(dinv[:, None] * (acc + y) + gcn_b[i][None, :], 0.2)

    hpad = jnp.zeros((NPAD, 64), jnp.float32).at[:N].set(node)
    out = pl.pallas_call(
        _post_kernel,
        grid=(NPAD // NB,),
        in_specs=[
            pl.BlockSpec((NB, 64), _row_block),
            pl.BlockSpec((64, 64), lambda i: (0, 0)),
            pl.BlockSpec((64,), lambda i: (0,)),
            pl.BlockSpec((64, 64), lambda i: (0, 0)),
            pl.BlockSpec((64,), lambda i: (0,)),
            pl.BlockSpec((64, 4), lambda i: (0, 0)),
            pl.BlockSpec((4,), lambda i: (0,)),
        ],
        out_specs=pl.BlockSpec((NB, 4), _row_block),
        out_shape=jax.ShapeDtypeStruct((NPAD, 4), jnp.float32),
    )(hpad, Wp1, bp1, Wp2, bp2, Wr, br)[:N]
    return out


# speedup vs baseline: 5.7986x; 3.0689x over previous
import functools
import jax
import jax.numpy as jnp
from jax import lax
from jax.experimental import pallas as pl
from jax.experimental.pallas import tpu as pltpu
from jax.experimental.pallas import tpu_sc as plsc

N = 50000
NPAD = 50176
NB = 512
NT = 3136
EB = 2048
D = 64
DH = 32


def _leaky(v, s):
    return jnp.where(v >= 0, v, s * v)


def _pre_kernel(x_ref, W1n_ref, b1n_ref, W2n_ref, b2n_ref, o_ref):
    x = jnp.nan_to_num(x_ref[...], nan=0.0)
    h = _leaky(jnp.dot(x, W1n_ref[...], preferred_element_type=jnp.float32)
               + b1n_ref[...][None, :], 0.2)
    h = _leaky(jnp.dot(h, W2n_ref[...], preferred_element_type=jnp.float32)
               + b2n_ref[...][None, :], 0.2)
    o_ref[...] = h


def _edge_kernel(E, ea_ref, W1e_ref, b1e_ref, W2e_ref, b2e_ref, o_ref):
    ea = jnp.nan_to_num(ea_ref[...], nan=0.0)
    h = _leaky(jnp.dot(ea, W1e_ref[...], preferred_element_type=jnp.float32)
               + b1e_ref[...][None, :], 0.2)
    w = _leaky(jnp.dot(h, W2e_ref[...], preferred_element_type=jnp.float32)
               + b2e_ref[...][None, :], 0.005)
    i = pl.program_id(0)
    gid = i * EB + jax.lax.broadcasted_iota(jnp.int32, (EB, 1), 0)
    o_ref[...] = jnp.where(gid < E, w, 0.0)


def _dinv_kernel(p_ref, o_ref):
    deg = p_ref[0, :, 0] + 1.0
    dinv = deg ** -0.5
    o_ref[:, 0] = jnp.where(jnp.isinf(dinv), 0.0, dinv)


def _t1_kernel(n_ref, dinv_ref, W_ref, y_ref):
    y = dinv_ref[...] * jnp.dot(n_ref[...], W_ref[...],
                                preferred_element_type=jnp.float32)
    y_ref[0] = y[:, :DH]
    y_ref[1] = y[:, DH:]


def _t2_kernel(acc_ref, y_ref, dinv_ref, b_ref, W_ref, o_ref):
    dinv = dinv_ref[...]
    b = b_ref[...]
    h0 = dinv * (acc_ref[0] + y_ref[0]) + b[None, :DH]
    h1 = dinv * (acc_ref[1] + y_ref[1]) + b[None, DH:]
    h = _leaky(jnp.concatenate([h0, h1], axis=1), 0.2)
    y = dinv * jnp.dot(h, W_ref[...], preferred_element_type=jnp.float32)
    o_ref[0] = y[:, :DH]
    o_ref[1] = y[:, DH:]


def _t3_kernel(acc_ref, y_ref, dinv_ref, b_ref, Wp1_ref, bp1_ref, Wp2_ref,
               bp2_ref, Wr_ref, br_ref, o_ref):
    dinv = dinv_ref[...]
    b = b_ref[...]
    h0 = dinv * (acc_ref[0] + y_ref[0]) + b[None, :DH]
    h1 = dinv * (acc_ref[1] + y_ref[1]) + b[None, DH:]
    h = _leaky(jnp.concatenate([h0, h1], axis=1), 0.2)
    h = _leaky(jnp.dot(h, Wp1_ref[...], preferred_element_type=jnp.float32)
               + bp1_ref[...][None, :], 0.2)
    h = _leaky(jnp.dot(h, Wp2_ref[...], preferred_element_type=jnp.float32)
               + bp2_ref[...][None, :], 0.2)
    o_ref[...] = jnp.dot(h, Wr_ref[...], preferred_element_type=jnp.float32) \
        + br_ref[...][None, :]


def _row_block(i):
    return (i, 0)


_MESH = plsc.VectorSubcoreMesh(core_axis_name="c", subcore_axis_name="s")
_IOTA = lambda: lax.iota(jnp.int32, 16)


def _zero_vmem(ref, nrows, ncol16):
    z = jnp.zeros((16,), jnp.float32)

    def body(i, _):
        for k in range(ncol16):
            ref[i, pl.ds(k * 16, 16)] = z
        return 0

    lax.fori_loop(0, nrows, body, 0)


def _make_scatter_kernel(erows):
    rpt = erows // 16
    kch = 40
    assert rpt % kch == 0

    @functools.partial(
        pl.kernel,
        out_type=jax.ShapeDtypeStruct((2, NPAD, DH), jnp.float32),
        mesh=_MESH,
        compiler_params=pltpu.CompilerParams(use_tc_tiling_on_sc=False),
        scratch_types=[
            pltpu.VMEM_SHARED((NPAD, DH), jnp.float32),
            pltpu.VMEM((kch, 128), jnp.int32),
            pltpu.VMEM((kch, 128), jnp.int32),
            pltpu.VMEM((kch, 128), jnp.float32),
            pltpu.VMEM((128, DH), jnp.float32),
            pltpu.VMEM((224, DH), jnp.float32),
            pltpu.SemaphoreType.DMA,
        ],
    )
    def scatter_kernel(y_hbm, r_hbm, c_hbm, w_hbm, out_hbm, acc, ridx, cidx,
                       wbuf, rows, zbuf, sem):
        cid = lax.axis_index("c")
        sid = lax.axis_index("s")
        _zero_vmem(zbuf, 224, DH // 16)
        for k in range(14):
            pltpu.sync_copy(zbuf, acc.at[pl.ds(sid * NT + k * 224, 224)])
        plsc.subcore_barrier()

        tbase = sid * rpt

        def blk_body(blk, _):
            base = tbase + blk * kch
            pltpu.sync_copy(r_hbm.at[pl.ds(base, kch)], ridx)
            pltpu.sync_copy(c_hbm.at[pl.ds(base, kch)], cidx)
            pltpu.sync_copy(w_hbm.at[pl.ds(base, kch)], wbuf)

            def row_body(j, _):
                pltpu.sync_copy(y_hbm.at[cid].at[ridx.at[j]], rows)

                def scale_body(g, _):
                    w16 = wbuf[j, pl.ds(g * 16, 16)]
                    for k in range(16):
                        ws = w16[jnp.full((16,), k, jnp.int32)]
                        e = g * 16 + k
                        v0 = rows[e, pl.ds(0, 16)] * ws
                        rows[e, pl.ds(0, 16)] = v0
                        v1 = rows[e, pl.ds(16, 16)] * ws
                        rows[e, pl.ds(16, 16)] = v1
                    return 0

                lax.fori_loop(0, 8, scale_body, 0)
                pltpu.sync_copy(rows, acc.at[cidx.at[j]], add=True)
                return 0

            lax.fori_loop(0, kch, row_body, 0)
            return 0

        lax.fori_loop(0, rpt // kch, blk_body, 0)
        plsc.subcore_barrier()
        pltpu.sync_copy(acc.at[pl.ds(sid * NT, NT)],
                        out_hbm.at[cid].at[pl.ds(sid * NT, NT)])

    return scatter_kernel


def kernel(x, edge_index, edge_attr, W1n, b1n, W2n, b2n, W1e, b1e, W2e, b2e,
           gcn_W, gcn_b, Wp1, bp1, Wp2, bp2, Wr, br):
    E = edge_index.shape[1]
    EPAD = ((E + 81919) // 81920) * 81920
    EROWS = EPAD // 128

    xpad = jnp.zeros((NPAD, 7), jnp.float32).at[:N].set(x)
    node = pl.pallas_call(
        _pre_kernel,
        grid=(NPAD // NB,),
        in_specs=[
            pl.BlockSpec((NB, 7), _row_block),
            pl.BlockSpec((7, D), lambda i: (0, 0)),
            pl.BlockSpec((D,), lambda i: (0,)),
            pl.BlockSpec((D, D), lambda i: (0, 0)),
            pl.BlockSpec((D,), lambda i: (0,)),
        ],
        out_specs=pl.BlockSpec((NB, D), _row_block),
        out_shape=jax.ShapeDtypeStruct((NPAD, D), jnp.float32),
    )(xpad, W1n, b1n, W2n, b2n)

    eapad = jnp.zeros((EPAD, 4), jnp.float32).at[:E].set(edge_attr)
    ew = pl.pallas_call(
        functools.partial(_edge_kernel, E),
        grid=(EPAD // EB,),
        in_specs=[
            pl.BlockSpec((EB, 4), _row_block),
            pl.BlockSpec((4, 16), lambda i: (0, 0)),
            pl.BlockSpec((16,), lambda i: (0,)),
            pl.BlockSpec((16, 1), lambda i: (0, 0)),
            pl.BlockSpec((1,), lambda i: (0,)),
        ],
        out_specs=pl.BlockSpec((EB, 1), _row_block),
        out_shape=jax.ShapeDtypeStruct((EPAD, 1), jnp.float32),
    )(eapad, W1e, b1e, W2e, b2e)

    w2d = ew.reshape(EROWS, 128)
    r2d = jnp.zeros((EPAD,), jnp.int32).at[:E].set(edge_index[0]).reshape(EROWS, 128)
    c2d = jnp.zeros((EPAD,), jnp.int32).at[:E].set(edge_index[1]).reshape(EROWS, 128)

    scatter = _make_scatter_kernel(EROWS)
    ones = jnp.ones((2, NPAD, DH), jnp.float32)
    degp = scatter(ones, r2d, c2d, w2d)

    dinv = pl.pallas_call(
        _dinv_kernel,
        grid=(NPAD // NB,),
        in_specs=[pl.BlockSpec((2, NB, DH), lambda i: (0, i, 0))],
        out_specs=pl.BlockSpec((NB, 1), _row_block),
        out_shape=jax.ShapeDtypeStruct((NPAD, 1), jnp.float32),
    )(degp)

    y = pl.pallas_call(
        _t1_kernel,
        grid=(NPAD // NB,),
        in_specs=[
            pl.BlockSpec((NB, D), _row_block),
            pl.BlockSpec((NB, 1), _row_block),
            pl.BlockSpec((D, D), lambda i: (0, 0)),
        ],
        out_specs=pl.BlockSpec((2, NB, DH), lambda i: (0, i, 0)),
        out_shape=jax.ShapeDtypeStruct((2, NPAD, DH), jnp.float32),
    )(node, dinv, gcn_W[0])

    for i in range(8):
        acc = scatter(y, r2d, c2d, w2d)
        if i < 7:
            y = pl.pallas_call(
                _t2_kernel,
                grid=(NPAD // NB,),
                in_specs=[
                    pl.BlockSpec((2, NB, DH), lambda i: (0, i, 0)),
                    pl.BlockSpec((2, NB, DH), lambda i: (0, i, 0)),
                    pl.BlockSpec((NB, 1), _row_block),
                    pl.BlockSpec((D,), lambda i: (0,)),
                    pl.BlockSpec((D, D), lambda i: (0, 0)),
                ],
                out_specs=pl.BlockSpec((2, NB, DH), lambda i: (0, i, 0)),
                out_shape=jax.ShapeDtypeStruct((2, NPAD, DH), jnp.float32),
            )(acc, y, dinv, gcn_b[i], gcn_W[i + 1])

    out = pl.pallas_call(
        _t3_kernel,
        grid=(NPAD // NB,),
        in_specs=[
            pl.BlockSpec((2, NB, DH), lambda i: (0, i, 0)),
            pl.BlockSpec((2, NB, DH), lambda i: (0, i, 0)),
            pl.BlockSpec((NB, 1), _row_block),
            pl.BlockSpec((D,), lambda i: (0,)),
            pl.BlockSpec((D, D), lambda i: (0, 0)),
            pl.BlockSpec((D,), lambda i: (0,)),
            pl.BlockSpec((D, D), lambda i: (0, 0)),
            pl.BlockSpec((D,), lambda i: (0,)),
            pl.BlockSpec((D, 4), lambda i: (0, 0)),
            pl.BlockSpec((4,), lambda i: (0,)),
        ],
        out_specs=pl.BlockSpec((NB, 4), _row_block),
        out_shape=jax.ShapeDtypeStruct((NPAD, 4), jnp.float32),
    )(acc, y, dinv, gcn_b[7], Wp1, bp1, Wp2, bp2, Wr, br)
    return out[:N]


# speedup vs baseline: 11.0182x; 1.9002x over previous
import functools
import jax
import jax.numpy as jnp
from jax import lax
from jax.experimental import pallas as pl
from jax.experimental.pallas import tpu as pltpu
from jax.experimental.pallas import tpu_sc as plsc

N = 50000
NPAD = 50176
NB = 512
NT = 3136
EB = 2048
D = 64
DH = 32


def _leaky(v, s):
    return jnp.where(v >= 0, v, s * v)


def _pre_kernel(x_ref, W1n_ref, b1n_ref, W2n_ref, b2n_ref, o_ref):
    x = jnp.nan_to_num(x_ref[...], nan=0.0)
    h = _leaky(jnp.dot(x, W1n_ref[...], preferred_element_type=jnp.float32)
               + b1n_ref[...][None, :], 0.2)
    h = _leaky(jnp.dot(h, W2n_ref[...], preferred_element_type=jnp.float32)
               + b2n_ref[...][None, :], 0.2)
    o_ref[...] = h


def _edge_kernel(E, ea_ref, W1e_ref, b1e_ref, W2e_ref, b2e_ref, o_ref):
    ea = jnp.nan_to_num(ea_ref[...], nan=0.0)
    h = _leaky(jnp.dot(ea, W1e_ref[...], preferred_element_type=jnp.float32)
               + b1e_ref[...][None, :], 0.2)
    w = _leaky(jnp.dot(h, W2e_ref[...], preferred_element_type=jnp.float32)
               + b2e_ref[...][None, :], 0.005)
    i = pl.program_id(0)
    gid = i * EB + jax.lax.broadcasted_iota(jnp.int32, (EB, 1), 0)
    o_ref[...] = jnp.where(gid < E, w, 0.0)


def _dinv_kernel(p_ref, o_ref):
    deg = p_ref[0, :, 0] + 1.0
    dinv = deg ** -0.5
    o_ref[:, 0] = jnp.where(jnp.isinf(dinv), 0.0, dinv)


def _t1_kernel(n_ref, dinv_ref, W_ref, y_ref):
    y = dinv_ref[...] * jnp.dot(n_ref[...], W_ref[...],
                                preferred_element_type=jnp.float32)
    y_ref[0] = y[:, :DH]
    y_ref[1] = y[:, DH:]


def _t2_kernel(acc_ref, y_ref, dinv_ref, b_ref, W_ref, o_ref):
    dinv = dinv_ref[...]
    b = b_ref[...]
    h0 = dinv * (acc_ref[0] + y_ref[0]) + b[None, :DH]
    h1 = dinv * (acc_ref[1] + y_ref[1]) + b[None, DH:]
    h = _leaky(jnp.concatenate([h0, h1], axis=1), 0.2)
    y = dinv * jnp.dot(h, W_ref[...], preferred_element_type=jnp.float32)
    o_ref[0] = y[:, :DH]
    o_ref[1] = y[:, DH:]


def _t3_kernel(acc_ref, y_ref, dinv_ref, b_ref, Wp1_ref, bp1_ref, Wp2_ref,
               bp2_ref, Wr_ref, br_ref, o_ref):
    dinv = dinv_ref[...]
    b = b_ref[...]
    h0 = dinv * (acc_ref[0] + y_ref[0]) + b[None, :DH]
    h1 = dinv * (acc_ref[1] + y_ref[1]) + b[None, DH:]
    h = _leaky(jnp.concatenate([h0, h1], axis=1), 0.2)
    h = _leaky(jnp.dot(h, Wp1_ref[...], preferred_element_type=jnp.float32)
               + bp1_ref[...][None, :], 0.2)
    h = _leaky(jnp.dot(h, Wp2_ref[...], preferred_element_type=jnp.float32)
               + bp2_ref[...][None, :], 0.2)
    o_ref[...] = jnp.dot(h, Wr_ref[...], preferred_element_type=jnp.float32) \
        + br_ref[...][None, :]


def _row_block(i):
    return (i, 0)


_MESH = plsc.VectorSubcoreMesh(core_axis_name="c", subcore_axis_name="s")
_IOTA = lambda: lax.iota(jnp.int32, 16)


def _zero_vmem(ref, nrows, ncol16):
    z = jnp.zeros((16,), jnp.float32)

    def body(i, _):
        for k in range(ncol16):
            ref[i, pl.ds(k * 16, 16)] = z
        return 0

    lax.fori_loop(0, nrows, body, 0)


def _make_scatter_kernel(egroups):
    gpt = egroups // 16
    assert gpt % 4 == 0 and gpt >= 8

    @functools.partial(
        pl.kernel,
        out_type=jax.ShapeDtypeStruct((2, NPAD, DH), jnp.float32),
        mesh=_MESH,
        compiler_params=pltpu.CompilerParams(use_tc_tiling_on_sc=False),
        scratch_types=[
            pltpu.VMEM_SHARED((NPAD, DH), jnp.float32),
            pltpu.VMEM((4, 2, 2, 128), jnp.int32),
            pltpu.VMEM((4, 2, 128), jnp.float32),
            pltpu.VMEM((2, 256, DH), jnp.float32),
            pltpu.VMEM((112, DH), jnp.float32),
            pltpu.SemaphoreType.DMA((4,)),
            pltpu.SemaphoreType.DMA((2,)),
            pltpu.SemaphoreType.DMA((2,)),
        ],
    )
    def scatter_kernel(y_hbm, e_hbm, w_hbm, out_hbm, acc, ebuf, wbuf, rows,
                       zbuf, semI, semG, semS):
        cid = lax.axis_index("c")
        sid = lax.axis_index("s")
        _zero_vmem(zbuf, 112, DH // 16)
        for k in range(28):
            pltpu.sync_copy(zbuf, acc.at[pl.ds(sid * NT + k * 112, 112)])
        plsc.subcore_barrier()

        g0 = sid * gpt
        yh = y_hbm.at[cid]

        def fire_edat(b, slot):
            pltpu.async_copy(e_hbm.at[g0 + b], ebuf.at[slot], semI.at[slot])
            pltpu.async_copy(w_hbm.at[g0 + b], wbuf.at[slot], semI.at[slot])

        def drain_edat(slot):
            pltpu.make_async_copy(e_hbm.at[0], ebuf.at[slot],
                                  semI.at[slot]).wait()
            pltpu.make_async_copy(w_hbm.at[0], wbuf.at[slot],
                                  semI.at[slot]).wait()

        def fire_gathers(slot, p):
            for j in range(2):
                pltpu.async_copy(yh.at[ebuf.at[slot].at[0].at[j]],
                                 rows.at[p].at[pl.ds(j * 128, 128)],
                                 semG.at[p])

        def drain_gathers(slot, p):
            for j in range(2):
                pltpu.make_async_copy(yh.at[ebuf.at[slot].at[0].at[j]],
                                      rows.at[p].at[pl.ds(j * 128, 128)],
                                      semG.at[p]).wait()

        def fire_scatters(slot, p):
            for j in range(2):
                pltpu.async_copy(rows.at[p].at[pl.ds(j * 128, 128)],
                                 acc.at[ebuf.at[slot].at[1].at[j]],
                                 semS.at[p], add=True)

        def drain_scatters(slot, p):
            for j in range(2):
                pltpu.make_async_copy(rows.at[p].at[pl.ds(j * 128, 128)],
                                      acc.at[ebuf.at[slot].at[1].at[j]],
                                      semS.at[p]).wait()

        def scale(slot, p):
            def sb(e16, _):
                jr = e16 // 8
                off = (e16 % 8) * 16
                w16 = wbuf[slot, jr, pl.ds(off, 16)]
                base = e16 * 16
                for k in range(16):
                    ws = w16[jnp.full((16,), k, jnp.int32)]
                    e = base + k
                    v0 = rows[p, e, pl.ds(0, 16)] * ws
                    rows[p, e, pl.ds(0, 16)] = v0
                    v1 = rows[p, e, pl.ds(16, 16)] * ws
                    rows[p, e, pl.ds(16, 16)] = v1
                return 0

            lax.fori_loop(0, 16, sb, 0)

        def stage(m, t, first, has1, has2):
            pm = t % 2
            pn = 1 - pm
            if not first:
                drain_scatters((t + 3) % 4, pn)
            if has1:
                drain_edat((t + 1) % 4)
            if has2:
                fire_edat(m + 2, (t + 2) % 4)
            if has1:
                fire_gathers((t + 1) % 4, pn)
            drain_gathers(t, pm)
            scale(t, pm)
            fire_scatters(t, pm)

        fire_edat(0, 0)
        fire_edat(1, 1)
        drain_edat(0)
        fire_gathers(0, 0)

        def body(q, _):
            mb = q * 4
            stage(mb + 0, 0, False, True, True)
            stage(mb + 1, 1, False, True, True)
            stage(mb + 2, 2, False, True, True)
            stage(mb + 3, 3, False, True, True)
            return 0

        stage(0, 0, True, True, True)
        stage(1, 1, False, True, True)
        stage(2, 2, False, True, True)
        stage(3, 3, False, True, True)
        lax.fori_loop(1, gpt // 4 - 1, body, 0)
        mb = gpt - 4
        stage(mb + 0, 0, False, True, True)
        stage(mb + 1, 1, False, True, True)
        stage(mb + 2, 2, False, True, False)
        stage(mb + 3, 3, False, False, False)
        drain_scatters(3, 1)
        plsc.subcore_barrier()
        pltpu.sync_copy(acc.at[pl.ds(sid * NT, NT)],
                        out_hbm.at[cid].at[pl.ds(sid * NT, NT)])

    return scatter_kernel


def kernel(x, edge_index, edge_attr, W1n, b1n, W2n, b2n, W1e, b1e, W2e, b2e,
           gcn_W, gcn_b, Wp1, bp1, Wp2, bp2, Wr, br):
    E = edge_index.shape[1]
    EPAD = ((E + 16383) // 16384) * 16384
    EROWS = EPAD // 128

    xpad = jnp.zeros((NPAD, 7), jnp.float32).at[:N].set(x)
    node = pl.pallas_call(
        _pre_kernel,
        grid=(NPAD // NB,),
        in_specs=[
            pl.BlockSpec((NB, 7), _row_block),
            pl.BlockSpec((7, D), lambda i: (0, 0)),
            pl.BlockSpec((D,), lambda i: (0,)),
            pl.BlockSpec((D, D), lambda i: (0, 0)),
            pl.BlockSpec((D,), lambda i: (0,)),
        ],
        out_specs=pl.BlockSpec((NB, D), _row_block),
        out_shape=jax.ShapeDtypeStruct((NPAD, D), jnp.float32),
    )(xpad, W1n, b1n, W2n, b2n)

    eapad = jnp.zeros((EPAD, 4), jnp.float32).at[:E].set(edge_attr)
    ew = pl.pallas_call(
        functools.partial(_edge_kernel, E),
        grid=(EPAD // EB,),
        in_specs=[
            pl.BlockSpec((EB, 4), _row_block),
            pl.BlockSpec((4, 16), lambda i: (0, 0)),
            pl.BlockSpec((16,), lambda i: (0,)),
            pl.BlockSpec((16, 1), lambda i: (0, 0)),
            pl.BlockSpec((1,), lambda i: (0,)),
        ],
        out_specs=pl.BlockSpec((EB, 1), _row_block),
        out_shape=jax.ShapeDtypeStruct((EPAD, 1), jnp.float32),
    )(eapad, W1e, b1e, W2e, b2e)

    w3 = ew.reshape(EROWS // 2, 2, 128)
    r3 = jnp.zeros((EPAD,), jnp.int32).at[:E].set(edge_index[0]).reshape(EROWS // 2, 2, 128)
    c3 = jnp.zeros((EPAD,), jnp.int32).at[:E].set(edge_index[1]).reshape(EROWS // 2, 2, 128)
    edat = jnp.stack([r3, c3], axis=1)

    scatter = _make_scatter_kernel(EROWS // 2)
    ones = jnp.ones((2, NPAD, DH), jnp.float32)
    degp = scatter(ones, edat, w3)

    dinv = pl.pallas_call(
        _dinv_kernel,
        grid=(NPAD // NB,),
        in_specs=[pl.BlockSpec((2, NB, DH), lambda i: (0, i, 0))],
        out_specs=pl.BlockSpec((NB, 1), _row_block),
        out_shape=jax.ShapeDtypeStruct((NPAD, 1), jnp.float32),
    )(degp)

    y = pl.pallas_call(
        _t1_kernel,
        grid=(NPAD // NB,),
        in_specs=[
            pl.BlockSpec((NB, D), _row_block),
            pl.BlockSpec((NB, 1), _row_block),
            pl.BlockSpec((D, D), lambda i: (0, 0)),
        ],
        out_specs=pl.BlockSpec((2, NB, DH), lambda i: (0, i, 0)),
        out_shape=jax.ShapeDtypeStruct((2, NPAD, DH), jnp.float32),
    )(node, dinv, gcn_W[0])

    for i in range(8):
        acc = scatter(y, edat, w3)
        if i < 7:
            y = pl.pallas_call(
                _t2_kernel,
                grid=(NPAD // NB,),
                in_specs=[
                    pl.BlockSpec((2, NB, DH), lambda i: (0, i, 0)),
                    pl.BlockSpec((2, NB, DH), lambda i: (0, i, 0)),
                    pl.BlockSpec((NB, 1), _row_block),
                    pl.BlockSpec((D,), lambda i: (0,)),
                    pl.BlockSpec((D, D), lambda i: (0, 0)),
                ],
                out_specs=pl.BlockSpec((2, NB, DH), lambda i: (0, i, 0)),
                out_shape=jax.ShapeDtypeStruct((2, NPAD, DH), jnp.float32),
            )(acc, y, dinv, gcn_b[i], gcn_W[i + 1])

    out = pl.pallas_call(
        _t3_kernel,
        grid=(NPAD // NB,),
        in_specs=[
            pl.BlockSpec((2, NB, DH), lambda i: (0, i, 0)),
            pl.BlockSpec((2, NB, DH), lambda i: (0, i, 0)),
            pl.BlockSpec((NB, 1), _row_block),
            pl.BlockSpec((D,), lambda i: (0,)),
            pl.BlockSpec((D, D), lambda i: (0, 0)),
            pl.BlockSpec((D,), lambda i: (0,)),
            pl.BlockSpec((D, D), lambda i: (0, 0)),
            pl.BlockSpec((D,), lambda i: (0,)),
            pl.BlockSpec((D, 4), lambda i: (0, 0)),
            pl.BlockSpec((4,), lambda i: (0,)),
        ],
        out_specs=pl.BlockSpec((NB, 4), _row_block),
        out_shape=jax.ShapeDtypeStruct((NPAD, 4), jnp.float32),
    )(acc, y, dinv, gcn_b[7], Wp1, bp1, Wp2, bp2, Wr, br)
    return out[:N]
